# baseline (speedup 1.0000x reference)
import functools

import jax
import jax.numpy as jnp
from jax import lax
from jax.experimental import pallas as pl
from jax.experimental.pallas import tpu as pltpu
from jax.experimental.pallas import tpu_sc as plsc

N_NODES = 10000
N_EDGES = 320000
NFEAT = 128
NHID = 128
NCLASS = 40

NC = 2
NS = 16
NW = NC * NS
EDGES_PER_TILE = N_EDGES // NW
DCHUNK = 80
DNCHUNK = EDGES_PER_TILE // DCHUNK
ROWS_PER_TILE = N_NODES // NS
ZR = 25

CHUNK = 80
NCHUNK = EDGES_PER_TILE // CHUNK
PAGES = 5
PCHUNK = NCHUNK // PAGES

_mesh = functools.partial(
    plsc.VectorSubcoreMesh, core_axis_name="c", subcore_axis_name="s")


def _zero_fill(zb, d):
    z16 = jnp.zeros((16,), jnp.float32)
    for r in range(ZR):
        for q in range(d // 16):
            zb[r, pl.ds(16 * q, 16)] = z16


def _zero_acc(zb, acc, s):
    for j in range(ROWS_PER_TILE // ZR):
        pltpu.sync_copy(zb, acc.at[pl.ds(s * ROWS_PER_TILE + j * ZR, ZR)])


def _copy_out(acc, out, c, s):
    pltpu.sync_copy(acc.at[pl.ds(s * ROWS_PER_TILE, ROWS_PER_TILE)],
                    out.at[c, s])


def _deg_body(dst_hbm, out_hbm, dst_v, ones_v, zb, acc, sem):
    c = lax.axis_index("c")
    s = lax.axis_index("s")
    wid = s * NC + c
    o16 = jnp.ones((16,), jnp.float32)
    for r in range(DCHUNK):
        for q in range(NHID // 16):
            ones_v[r, pl.ds(16 * q, 16)] = o16
    _zero_fill(zb, NHID)
    _zero_acc(zb, acc, s)
    pltpu.sync_copy(dst_hbm.at[wid], dst_v)
    plsc.subcore_barrier()

    def body(i, carry):
        pltpu.async_copy(ones_v, acc.at[dst_v.at[i]], sem, add=True)
        return carry

    lax.fori_loop(0, DNCHUNK, body, 0)

    def drain(i, carry):
        pltpu.make_async_copy(ones_v, acc.at[dst_v.at[i]], sem).wait()
        return carry

    lax.fori_loop(0, DNCHUNK, drain, 0)
    plsc.subcore_barrier()
    _copy_out(acc, out_hbm, c, s)


def _deg(dst3):
    kern = pl.kernel(
        _deg_body,
        out_type=jax.ShapeDtypeStruct((NC, NS, ROWS_PER_TILE, NHID),
                                      jnp.float32),
        mesh=_mesh(),
        scratch_types=[
            pltpu.VMEM((DNCHUNK, DCHUNK), jnp.int32),
            pltpu.VMEM((DCHUNK, NHID), jnp.float32),
            pltpu.VMEM((ZR, NHID), jnp.float32),
            pltpu.VMEM_SHARED((N_NODES, NHID), jnp.float32),
            pltpu.SemaphoreType.DMA,
        ],
    )
    return kern(dst3).reshape(NC, N_NODES, NHID)


def _spmm_body(d, feat_hbm, src_hbm, dst_hbm, out_hbm,
               src_pg, dst_pg, rows0, rows1, rows2, acc,
               sr0, sr1, sr2, sw0, sw1, sw2):
    c = lax.axis_index("c")
    s = lax.axis_index("s")
    wid = s * NC + c
    rows = [rows0, rows1, rows2]
    srs = [sr0, sr1, sr2]
    sws = [sw0, sw1, sw2]
    z16 = jnp.zeros((16,), jnp.float32)
    for r in range(ZR):
        for q in range(d // 16):
            rows0[r, pl.ds(16 * q, 16)] = z16
    for j in range(ROWS_PER_TILE // ZR):
        pltpu.sync_copy(rows0.at[pl.ds(0, ZR)],
                        acc.at[pl.ds(s * ROWS_PER_TILE + j * ZR, ZR)])
    plsc.subcore_barrier()

    def gath(i, b):
        pltpu.make_async_copy(
            feat_hbm.at[src_pg.at[i]], rows[b], srs[b]).start()

    def gath_wait(i, b):
        pltpu.make_async_copy(
            feat_hbm.at[src_pg.at[i]], rows[b], srs[b]).wait()

    def scat(i, b):
        pltpu.async_copy(rows[b], acc.at[dst_pg.at[i]], sws[b], add=True)

    def scat_wait(i, b):
        pltpu.make_async_copy(rows[b], acc.at[dst_pg.at[i]], sws[b]).wait()

    def step(i, unsafe_py_j):
        b = unsafe_py_j % 3
        gath_wait(i, b)
        scat(i, b)
        if unsafe_py_j >= 1:
            scat_wait(i - 1, (unsafe_py_j + 2) % 3)
        if unsafe_py_j <= PCHUNK - 3:
            gath(i + 2, (unsafe_py_j + 2) % 3)

    for p in range(PAGES):
        pltpu.sync_copy(src_hbm.at[wid, p], src_pg)
        pltpu.sync_copy(dst_hbm.at[wid, p], dst_pg)
        gath(0, 0)
        gath(1, 1)
        for i in range(PCHUNK):
            step(i, i)
        scat_wait(PCHUNK - 1, (PCHUNK - 1) % 3)

    plsc.subcore_barrier()
    _copy_out(acc, out_hbm, c, s)


def _spmm(feat, srcp, dstp, d):
    kern = pl.kernel(
        functools.partial(_spmm_body, d),
        out_type=jax.ShapeDtypeStruct((NC, NS, ROWS_PER_TILE, d),
                                      jnp.float32),
        mesh=_mesh(),
        scratch_types=[
            pltpu.VMEM((PCHUNK, CHUNK), jnp.int32),
            pltpu.VMEM((PCHUNK, CHUNK), jnp.int32),
            pltpu.VMEM((CHUNK, d), jnp.float32),
            pltpu.VMEM((CHUNK, d), jnp.float32),
            pltpu.VMEM((CHUNK, d), jnp.float32),
            pltpu.VMEM_SHARED((N_NODES, d), jnp.float32),
            pltpu.SemaphoreType.DMA,
            pltpu.SemaphoreType.DMA,
            pltpu.SemaphoreType.DMA,
            pltpu.SemaphoreType.DMA,
            pltpu.SemaphoreType.DMA,
            pltpu.SemaphoreType.DMA,
        ],
    )
    return kern(feat, srcp, dstp).reshape(NC, N_NODES, d)


ROWS_TC = 2000


def _matmul_body(x_ref, w1_ref, xw_ref):
    xw_ref[...] = jnp.dot(x_ref[...], w1_ref[...],
                          preferred_element_type=jnp.float32)


def _scale_in_body(xw_ref, degc_ref, xws_ref, dinv_ref):
    deg = jnp.maximum(degc_ref[0] + degc_ref[1], 1.0)
    dinv = lax.rsqrt(deg)
    dinv_ref[...] = dinv
    xws_ref[...] = xw_ref[...] * dinv[:, 0:1]


def _mid_body(p_ref, dinv_ref, out_ref):
    dv = dinv_ref[:, 0:1]
    h = jnp.maximum((p_ref[0] + p_ref[1]) * dv, 0.0)
    out_ref[...] = h * dv


def _final_body(q_ref, dinv_ref, w2_ref, b2_ref, out_ref):
    z = (q_ref[0] + q_ref[1]) * dinv_ref[:, 0:1]
    logits = jnp.dot(z, w2_ref[...],
                     preferred_element_type=jnp.float32) + b2_ref[0:1, :]
    mx = jnp.max(logits, axis=1, keepdims=True)
    lse = jnp.log(jnp.sum(jnp.exp(logits - mx), axis=1, keepdims=True)) + mx
    out_ref[...] = logits - lse


def kernel(x, edge_index, W1, W2, b2):
    ei = edge_index.astype(jnp.int32)
    dst3 = ei[1].reshape(NW, DNCHUNK, DCHUNK)
    srcp = ei[0].reshape(NW, PAGES, PCHUNK, CHUNK)
    dstp = ei[1].reshape(NW, PAGES, PCHUNK, CHUNK)
    b2r = b2.reshape(1, NCLASS)

    grid = (N_NODES // ROWS_TC,)
    xw = pl.pallas_call(
        _matmul_body,
        grid=grid,
        in_specs=[
            pl.BlockSpec((ROWS_TC, NFEAT), lambda i: (i, 0)),
            pl.BlockSpec((NFEAT, NHID), lambda i: (0, 0)),
        ],
        out_specs=pl.BlockSpec((ROWS_TC, NHID), lambda i: (i, 0)),
        out_shape=jax.ShapeDtypeStruct((N_NODES, NHID), jnp.float32),
    )(x, W1)

    degc = _deg(dst3)[:, :, :16]

    xws, dinv16 = pl.pallas_call(
        _scale_in_body,
        grid=grid,
        in_specs=[
            pl.BlockSpec((ROWS_TC, NHID), lambda i: (i, 0)),
            pl.BlockSpec((NC, ROWS_TC, 16), lambda i: (0, i, 0)),
        ],
        out_specs=[
            pl.BlockSpec((ROWS_TC, NHID), lambda i: (i, 0)),
            pl.BlockSpec((ROWS_TC, 16), lambda i: (i, 0)),
        ],
        out_shape=[
            jax.ShapeDtypeStruct((N_NODES, NHID), jnp.float32),
            jax.ShapeDtypeStruct((N_NODES, 16), jnp.float32),
        ],
    )(xw, degc)

    p = _spmm(xws, srcp, dstp, NHID)

    hs = pl.pallas_call(
        _mid_body,
        grid=grid,
        in_specs=[
            pl.BlockSpec((NC, ROWS_TC, NHID), lambda i: (0, i, 0)),
            pl.BlockSpec((ROWS_TC, 16), lambda i: (i, 0)),
        ],
        out_specs=pl.BlockSpec((ROWS_TC, NHID), lambda i: (i, 0)),
        out_shape=jax.ShapeDtypeStruct((N_NODES, NHID), jnp.float32),
    )(p, dinv16)

    q = _spmm(hs, srcp, dstp, NHID)

    out = pl.pallas_call(
        _final_body,
        grid=grid,
        in_specs=[
            pl.BlockSpec((NC, ROWS_TC, NHID), lambda i: (0, i, 0)),
            pl.BlockSpec((ROWS_TC, 16), lambda i: (i, 0)),
            pl.BlockSpec((NHID, NCLASS), lambda i: (0, 0)),
            pl.BlockSpec((1, NCLASS), lambda i: (0, 0)),
        ],
        out_specs=pl.BlockSpec((ROWS_TC, NCLASS), lambda i: (i, 0)),
        out_shape=jax.ShapeDtypeStruct((N_NODES, NCLASS), jnp.float32),
    )(q, dinv16, W2, b2r)
    return out


# speedup vs baseline: 1.0024x; 1.0024x over previous
import functools

import jax
import jax.numpy as jnp
from jax import lax
from jax.experimental import pallas as pl
from jax.experimental.pallas import tpu as pltpu
from jax.experimental.pallas import tpu_sc as plsc

N_NODES = 10000
N_EDGES = 320000
NFEAT = 128
NHID = 128
NCLASS = 40

NC = 2
NS = 16
NW = NC * NS
EDGES_PER_TILE = N_EDGES // NW
DCHUNK = 80
DNCHUNK = EDGES_PER_TILE // DCHUNK
ROWS_PER_TILE = N_NODES // NS
ZR = 25

CHUNK = 80
NCHUNK = EDGES_PER_TILE // CHUNK
PAGES = 5
PCHUNK = NCHUNK // PAGES

_mesh = functools.partial(
    plsc.VectorSubcoreMesh, core_axis_name="c", subcore_axis_name="s")


def _zero_fill(zb, d):
    z16 = jnp.zeros((16,), jnp.float32)
    for r in range(ZR):
        for q in range(d // 16):
            zb[r, pl.ds(16 * q, 16)] = z16


def _zero_acc(zb, acc, s):
    for j in range(ROWS_PER_TILE // ZR):
        pltpu.sync_copy(zb, acc.at[pl.ds(s * ROWS_PER_TILE + j * ZR, ZR)])


def _copy_out(acc, out, c, s):
    pltpu.sync_copy(acc.at[pl.ds(s * ROWS_PER_TILE, ROWS_PER_TILE)],
                    out.at[c, s])


def _deg_body(dst_hbm, out_hbm, dst_v, ones_v, zb, acc, sem):
    c = lax.axis_index("c")
    s = lax.axis_index("s")
    wid = s * NC + c
    o16 = jnp.ones((16,), jnp.float32)
    for r in range(DCHUNK):
        for q in range(NHID // 16):
            ones_v[r, pl.ds(16 * q, 16)] = o16
    _zero_fill(zb, NHID)
    _zero_acc(zb, acc, s)
    pltpu.sync_copy(dst_hbm.at[wid], dst_v)
    plsc.subcore_barrier()

    def body(i, carry):
        pltpu.async_copy(ones_v, acc.at[dst_v.at[i]], sem, add=True)
        return carry

    lax.fori_loop(0, DNCHUNK, body, 0)

    def drain(i, carry):
        pltpu.make_async_copy(ones_v, acc.at[dst_v.at[i]], sem).wait()
        return carry

    lax.fori_loop(0, DNCHUNK, drain, 0)
    plsc.subcore_barrier()
    _copy_out(acc, out_hbm, c, s)


def _deg(dst3):
    kern = pl.kernel(
        _deg_body,
        out_type=jax.ShapeDtypeStruct((NC, NS, ROWS_PER_TILE, NHID),
                                      jnp.float32),
        mesh=_mesh(),
        scratch_types=[
            pltpu.VMEM((DNCHUNK, DCHUNK), jnp.int32),
            pltpu.VMEM((DCHUNK, NHID), jnp.float32),
            pltpu.VMEM((ZR, NHID), jnp.float32),
            pltpu.VMEM_SHARED((N_NODES, NHID), jnp.float32),
            pltpu.SemaphoreType.DMA,
        ],
    )
    return kern(dst3).reshape(NC, N_NODES, NHID)


def _spmm_body(d, feat_hbm, src_hbm, dst_hbm, out_hbm,
               src_pg, dst_pg, rows0, rows1, rows2, acc,
               sr0, sr1, sr2, sw0, sw1, sw2):
    c = lax.axis_index("c")
    s = lax.axis_index("s")
    wid = s * NC + c
    rows = [rows0, rows1, rows2]
    srs = [sr0, sr1, sr2]
    sws = [sw0, sw1, sw2]
    z16 = jnp.zeros((16,), jnp.float32)
    for r in range(ZR):
        for q in range(d // 16):
            rows0[r, pl.ds(16 * q, 16)] = z16
    for j in range(ROWS_PER_TILE // ZR):
        pltpu.sync_copy(rows0.at[pl.ds(0, ZR)],
                        acc.at[pl.ds(s * ROWS_PER_TILE + j * ZR, ZR)])
    plsc.subcore_barrier()

    def gath(i, b):
        pltpu.make_async_copy(
            feat_hbm.at[src_pg.at[i]], rows[b], srs[b]).start()

    def gath_wait(i, b):
        pltpu.make_async_copy(
            feat_hbm.at[src_pg.at[i]], rows[b], srs[b]).wait()

    def scat(i, b):
        pltpu.async_copy(rows[b], acc.at[dst_pg.at[i]], sws[b], add=True)

    def scat_wait(i, b):
        pltpu.make_async_copy(rows[b], acc.at[dst_pg.at[i]], sws[b]).wait()

    def step(i, unsafe_py_j):
        b = unsafe_py_j % 3
        gath_wait(i, b)
        scat(i, b)
        if unsafe_py_j >= 1:
            scat_wait(i - 1, (unsafe_py_j + 2) % 3)
        if unsafe_py_j <= PCHUNK - 3:
            gath(i + 2, (unsafe_py_j + 2) % 3)

    for p in range(PAGES):
        pltpu.sync_copy(src_hbm.at[wid, p], src_pg)
        pltpu.sync_copy(dst_hbm.at[wid, p], dst_pg)
        gath(0, 0)
        gath(1, 1)
        for i in range(PCHUNK):
            step(i, i)
        scat_wait(PCHUNK - 1, (PCHUNK - 1) % 3)

    plsc.subcore_barrier()
    _copy_out(acc, out_hbm, c, s)


def _spmm(feat, srcp, dstp, d):
    kern = pl.kernel(
        functools.partial(_spmm_body, d),
        out_type=jax.ShapeDtypeStruct((NC, NS, ROWS_PER_TILE, d),
                                      jnp.float32),
        mesh=_mesh(),
        scratch_types=[
            pltpu.VMEM((PCHUNK, CHUNK), jnp.int32),
            pltpu.VMEM((PCHUNK, CHUNK), jnp.int32),
            pltpu.VMEM((CHUNK, d), jnp.float32),
            pltpu.VMEM((CHUNK, d), jnp.float32),
            pltpu.VMEM((CHUNK, d), jnp.float32),
            pltpu.VMEM_SHARED((N_NODES, d), jnp.float32),
            pltpu.SemaphoreType.DMA,
            pltpu.SemaphoreType.DMA,
            pltpu.SemaphoreType.DMA,
            pltpu.SemaphoreType.DMA,
            pltpu.SemaphoreType.DMA,
            pltpu.SemaphoreType.DMA,
        ],
    )
    return kern(feat, srcp, dstp).reshape(NC, N_NODES, d)


ROWS_TC = 2000


def _scale_in_body(x_ref, w1_ref, degc_ref, xws_ref, dinv_ref):
    deg = jnp.maximum(degc_ref[0] + degc_ref[1], 1.0)
    dinv = lax.rsqrt(deg)
    dinv_ref[...] = dinv
    xw = jnp.dot(x_ref[...], w1_ref[...],
                 preferred_element_type=jnp.float32)
    xws_ref[...] = xw * dinv[:, 0:1]


def _mid_body(p_ref, dinv_ref, out_ref):
    dv = dinv_ref[:, 0:1]
    h = jnp.maximum((p_ref[0] + p_ref[1]) * dv, 0.0)
    out_ref[...] = h * dv


def _final_body(q_ref, dinv_ref, w2_ref, b2_ref, out_ref):
    z = (q_ref[0] + q_ref[1]) * dinv_ref[:, 0:1]
    logits = jnp.dot(z, w2_ref[...],
                     preferred_element_type=jnp.float32) + b2_ref[0:1, :]
    mx = jnp.max(logits, axis=1, keepdims=True)
    lse = jnp.log(jnp.sum(jnp.exp(logits - mx), axis=1, keepdims=True)) + mx
    out_ref[...] = logits - lse


def kernel(x, edge_index, W1, W2, b2):
    ei = edge_index.astype(jnp.int32)
    dst3 = ei[1].reshape(NW, DNCHUNK, DCHUNK)
    srcp = ei[0].reshape(NW, PAGES, PCHUNK, CHUNK)
    dstp = ei[1].reshape(NW, PAGES, PCHUNK, CHUNK)
    b2r = b2.reshape(1, NCLASS)

    degc = _deg(dst3)[:, :, :16]

    grid = (N_NODES // ROWS_TC,)
    xws, dinv16 = pl.pallas_call(
        _scale_in_body,
        grid=grid,
        in_specs=[
            pl.BlockSpec((ROWS_TC, NFEAT), lambda i: (i, 0)),
            pl.BlockSpec((NFEAT, NHID), lambda i: (0, 0)),
            pl.BlockSpec((NC, ROWS_TC, 16), lambda i: (0, i, 0)),
        ],
        out_specs=[
            pl.BlockSpec((ROWS_TC, NHID), lambda i: (i, 0)),
            pl.BlockSpec((ROWS_TC, 16), lambda i: (i, 0)),
        ],
        out_shape=[
            jax.ShapeDtypeStruct((N_NODES, NHID), jnp.float32),
            jax.ShapeDtypeStruct((N_NODES, 16), jnp.float32),
        ],
    )(x, W1, degc)

    p = _spmm(xws, srcp, dstp, NHID)

    hs = pl.pallas_call(
        _mid_body,
        grid=grid,
        in_specs=[
            pl.BlockSpec((NC, ROWS_TC, NHID), lambda i: (0, i, 0)),
            pl.BlockSpec((ROWS_TC, 16), lambda i: (i, 0)),
        ],
        out_specs=pl.BlockSpec((ROWS_TC, NHID), lambda i: (i, 0)),
        out_shape=jax.ShapeDtypeStruct((N_NODES, NHID), jnp.float32),
    )(p, dinv16)

    q = _spmm(hs, srcp, dstp, NHID)

    out = pl.pallas_call(
        _final_body,
        grid=grid,
        in_specs=[
            pl.BlockSpec((NC, ROWS_TC, NHID), lambda i: (0, i, 0)),
            pl.BlockSpec((ROWS_TC, 16), lambda i: (i, 0)),
            pl.BlockSpec((NHID, NCLASS), lambda i: (0, 0)),
            pl.BlockSpec((1, NCLASS), lambda i: (0, 0)),
        ],
        out_specs=pl.BlockSpec((ROWS_TC, NCLASS), lambda i: (i, 0)),
        out_shape=jax.ShapeDtypeStruct((N_NODES, NCLASS), jnp.float32),
    )(q, dinv16, W2, b2r)
    return out
